# Initial kernel scaffold; baseline (speedup 1.0000x reference)
import jax
import jax.numpy as jnp
from jax.experimental import pallas as pl


def kernel(x, edge_index, batch, params):
    raise NotImplementedError("write your pallas kernel here")


# speedup vs baseline: 4.7422x; 4.7422x over previous
import functools

import jax
import jax.numpy as jnp
from jax import lax
from jax.experimental import pallas as pl
from jax.experimental.pallas import tpu as pltpu
from jax.experimental.pallas import tpu_sc as plsc

N = 10000
D = 128
H = 128
G = 64
E = 320000

NSC = 2
NTILE = 16
NW = NSC * NTILE
CH = 128
EP_CH = 79
EP = EP_CH * CH
ETOT = NW * EP
NPAD = 10240
RPT = NPAD // NTILE
ZR = 64

_EPS = 1e-5


DH = 64


def _sc_partials(h0, h1, src_g, dst_g):
  @functools.partial(
      pl.kernel,
      out_type=jax.ShapeDtypeStruct((NSC * 2 * NPAD, DH), jnp.float32),
      mesh=plsc.VectorSubcoreMesh(core_axis_name="c", subcore_axis_name="s"),
      compiler_params=pltpu.CompilerParams(use_tc_tiling_on_sc=False),
      scratch_types=[
          pltpu.VMEM((EP_CH, CH), jnp.int32),
          pltpu.VMEM((EP_CH, CH), jnp.int32),
          pltpu.VMEM((CH, DH), jnp.float32),
          pltpu.VMEM((CH, DH), jnp.float32),
          pltpu.VMEM((ZR, DH), jnp.float32),
          pltpu.VMEM_SHARED((NPAD, DH), jnp.float32),
          pltpu.SemaphoreType.DMA,
          pltpu.SemaphoreType.DMA,
      ],
  )
  def k(h0_hbm, h1_hbm, src_hbm, dst_hbm, out_hbm, src_v, dst_v, rows0,
        rows1, z_v, acc_sh, sem0, sem1):
    c = lax.axis_index("c")
    s = lax.axis_index("s")
    tile = c * NTILE + s

    def zrow(i, carry):
      for kk in range(DH // 16):
        z_v[i, pl.ds(kk * 16, 16)] = jnp.zeros((16,), jnp.float32)
      return carry

    lax.fori_loop(0, ZR, zrow, 0)

    pltpu.sync_copy(src_hbm.at[tile], src_v)
    pltpu.sync_copy(dst_hbm.at[tile], dst_v)

    for half in range(2):
      h_hbm = (h0_hbm, h1_hbm)[half]

      def zblk(j, carry):
        pltpu.sync_copy(z_v, acc_sh.at[pl.ds(s * RPT + j * ZR, ZR)])
        return carry

      lax.fori_loop(0, RPT // ZR, zblk, 0)

      plsc.subcore_barrier()

      def chunk2(jj, carry):
        j0 = jj * 2

        pltpu.make_async_copy(h_hbm.at[src_v.at[j0]], rows0, sem0).wait()
        pltpu.sync_copy(rows0, acc_sh.at[dst_v.at[j0]], add=True)

        @pl.when(j0 + 2 < EP_CH)
        def _():
          pltpu.async_copy(h_hbm.at[src_v.at[j0 + 2]], rows0, sem0)

        @pl.when(j0 + 1 < EP_CH)
        def _():
          pltpu.make_async_copy(h_hbm.at[src_v.at[j0 + 1]], rows1, sem1).wait()
          pltpu.sync_copy(rows1, acc_sh.at[dst_v.at[j0 + 1]], add=True)

        @pl.when(j0 + 3 < EP_CH)
        def _():
          pltpu.async_copy(h_hbm.at[src_v.at[j0 + 3]], rows1, sem1)

        return carry

      pltpu.async_copy(h_hbm.at[src_v.at[0]], rows0, sem0)
      pltpu.async_copy(h_hbm.at[src_v.at[1]], rows1, sem1)
      lax.fori_loop(0, (EP_CH + 1) // 2, chunk2, 0)

      plsc.subcore_barrier()
      pltpu.sync_copy(
          acc_sh.at[pl.ds(s * RPT, RPT)],
          out_hbm.at[pl.ds((c * 2 + half) * NPAD + s * RPT, RPT)])

  return k(h0, h1, src_g, dst_g)


def _bn_relu(y, g, b):
  m = jnp.mean(y, axis=0, keepdims=True)
  yc = y - m
  v = jnp.mean(yc * yc, axis=0, keepdims=True)
  return jnp.maximum(g * yc / jnp.sqrt(v + _EPS) + b, 0.0)


def _dot(a, b, precision=jax.lax.Precision.DEFAULT):
  return jax.lax.dot_general(
      a, b, (((1,), (0,)), ((), ())),
      precision=precision, preferred_element_type=jnp.float32)


def _agg(x_r, q0_r, q1_r, q2_r, q3_r):
  return x_r[...] + jnp.concatenate(
      [q0_r[...] + q2_r[...], q1_r[...] + q3_r[...]], axis=1)


def _mlp_kernel(x_r, q0_r, q1_r, q2_r, q3_r, w1_r, b1_r, g1_r, be1_r, w2_r,
                b2_r, g2_r, be2_r, out_r):
  z = _agg(x_r, q0_r, q1_r, q2_r, q3_r)
  y1 = _dot(z, w1_r[...]) + b1_r[...]
  z1 = _bn_relu(y1, g1_r[...], be1_r[...])
  y2 = _dot(z1, w2_r[...]) + b2_r[...]
  out_r[...] = _bn_relu(y2, g2_r[...], be2_r[...])


def _mlp_pool_kernel(x_r, q0_r, q1_r, q2_r, q3_r, w1_r, b1_r, g1_r, be1_r,
                     w2_r, b2_r, g2_r, be2_r, batch_r, wc1_r, bc1_r, gc_r,
                     bec_r, wc2_r, bc2_r, out_r, pooled_r, logits_r):
  z = _agg(x_r, q0_r, q1_r, q2_r, q3_r)
  y1 = _dot(z, w1_r[...]) + b1_r[...]
  z1 = _bn_relu(y1, g1_r[...], be1_r[...])
  y2 = _dot(z1, w2_r[...]) + b2_r[...]
  z2 = _bn_relu(y2, g2_r[...], be2_r[...])
  out_r[...] = z2

  gid = lax.broadcasted_iota(jnp.int32, (G, N), 0)
  onehot = (gid == batch_r[...]).astype(jnp.float32)
  sums = _dot(onehot, z2, precision=jax.lax.Precision.HIGHEST)
  cnt = jnp.sum(onehot, axis=1, keepdims=True)
  pooled = sums / jnp.maximum(cnt, 1.0)
  pooled_r[...] = pooled

  hh = _dot(pooled, wc1_r[...]) + bc1_r[...]
  hh = _bn_relu(hh, gc_r[...], bec_r[...])
  logits_r[...] = _dot(hh, wc2_r[...]) + bc2_r[...]


def kernel(x, edge_index, batch, params):
  pad = ETOT - E
  src_g = jnp.concatenate(
      [edge_index[0], jnp.zeros((pad,), jnp.int32)]).reshape(NW, EP_CH, CH)
  dst_g = jnp.concatenate(
      [edge_index[1], jnp.full((pad,), N, jnp.int32)]).reshape(NW, EP_CH, CH)

  c1, c2 = params['c1'], params['c2']
  row = lambda v: v.reshape(1, -1)

  mlp = pl.pallas_call(
      _mlp_kernel,
      out_shape=jax.ShapeDtypeStruct((N, H), jnp.float32),
  )
  wc2p = jnp.zeros((64, 128), jnp.float32).at[:, :2].set(params['Wc2'])
  bc2p = jnp.zeros((1, 128), jnp.float32).at[0, :2].set(params['bc2'])

  mlp_pool = pl.pallas_call(
      _mlp_pool_kernel,
      out_shape=[
          jax.ShapeDtypeStruct((N, H), jnp.float32),
          jax.ShapeDtypeStruct((G, H), jnp.float32),
          jax.ShapeDtypeStruct((G, 128), jnp.float32),
      ],
  )

  quarters = lambda p: tuple(p[i * NPAD:i * NPAD + N] for i in range(4))

  p1 = _sc_partials(x[:, :DH], x[:, DH:], src_g, dst_g)
  out1 = mlp(x, *quarters(p1),
             c1['W1'], row(c1['b1']), row(c1['g1']), row(c1['be1']),
             c1['W2'], row(c1['b2']), row(c1['g2']), row(c1['be2']))

  p2 = _sc_partials(out1[:, :DH], out1[:, DH:], src_g, dst_g)
  out2, pooled, logits_p = mlp_pool(
      out1, *quarters(p2),
      c2['W1'], row(c2['b1']), row(c2['g1']), row(c2['be1']),
      c2['W2'], row(c2['b2']), row(c2['g2']), row(c2['be2']),
      batch.reshape(1, N),
      params['Wc1'], row(params['bc1']), row(params['gc']), row(params['bec']),
      wc2p, bc2p)

  logits = logits_p[:, :2]
  return (logits, {'conv1': out1, 'conv2': out2, 'pooled': pooled})
